# baseline (speedup 1.0000x reference)
import functools

import jax
import jax.numpy as jnp
from jax import lax
from jax.experimental import pallas as pl
from jax.experimental.pallas import tpu as pltpu
from jax.experimental.pallas import tpu_sc as plsc

_NC, _NS, _L = 2, 16, 16
_NW = _NC * _NS

_N = 10000
_E = 320000
_D = 128

_EPT = _E // _NW
_K = 125
_CH = 80
_G = 16
_NG = _CH // _G
_NP = 10112
_RPS = _NP // _NS
_TRASH = _N + 8

_NPAD = 16384
_EPT1 = _E // _NS
_DPS = _NPAD // _NS


def _zero_rows(ref, nrows):
    z = jnp.zeros((_L,), jnp.float32)

    def body(r, carry):
        def inner(c, carry2):
            ref[r, pl.ds(c * _L, _L)] = z
            return carry2
        return lax.fori_loop(0, _D // _L, inner, carry)

    lax.fori_loop(0, nrows, body, 0)


def _rsqrt16(d):
    i = plsc.bitcast(d, jnp.int32)
    i = 0x5F3759DF - (i >> 1)
    y = plsc.bitcast(i, jnp.float32)
    h = 0.5 * d
    y = y * (1.5 - h * y * y)
    y = y * (1.5 - h * y * y)
    y = y * (1.5 - h * y * y)
    return y


def _dinv_body(dst_hbm, dinv_hbm, dstv, degv, redv, acc_sh):
    s = lax.axis_index("s")
    z = jnp.zeros((_L,), jnp.float32)

    def zz(i, carry):
        degv[pl.ds(i * _L, _L)] = z
        return carry

    lax.fori_loop(0, _NPAD // _L, zz, 0)
    pltpu.sync_copy(dst_hbm.at[s], dstv)

    ones = jnp.full((_L,), 1.0, jnp.float32)

    def hist(i, carry):
        idx = dstv[pl.ds(i * _L, _L)]
        plsc.addupdate_scatter(degv, [idx], ones)
        return carry

    lax.fori_loop(0, _EPT1 // _L, hist, 0)
    pltpu.sync_copy(degv, acc_sh.at[s])
    plsc.subcore_barrier()
    base = s * _DPS
    for t in range(_NS):
        pltpu.sync_copy(acc_sh.at[t, pl.ds(base, _DPS)], redv.at[t])

    def red(i, carry):
        acc = redv[0, pl.ds(i * _L, _L)]
        for t in range(1, _NS):
            acc = acc + redv[t, pl.ds(i * _L, _L)]
        degv[pl.ds(i * _L, _L)] = _rsqrt16(acc + 1.0)
        return carry

    lax.fori_loop(0, _DPS // _L, red, 0)
    pltpu.sync_copy(degv.at[pl.ds(0, _DPS)], dinv_hbm.at[pl.ds(base, _DPS)])


_dinv_call = functools.partial(
    pl.kernel,
    out_type=jax.ShapeDtypeStruct((_NPAD,), jnp.float32),
    mesh=plsc.VectorSubcoreMesh(core_axis_name="c", subcore_axis_name="s",
                                num_cores=1),
    scratch_types=[
        pltpu.VMEM((_EPT1,), jnp.int32),
        pltpu.VMEM((_NPAD,), jnp.float32),
        pltpu.VMEM((_NS, _DPS), jnp.float32),
        pltpu.VMEM_SHARED((_NS, _NPAD), jnp.float32),
    ],
    compiler_params=pltpu.CompilerParams(needs_layout_passes=False),
)(_dinv_body)


def _scatter_body(y_hbm, src_hbm, dst_hbm, out_hbm, srcv, dstv, rows, gsem,
                  acc_sh):
    c = lax.axis_index("c")
    s = lax.axis_index("s")
    wid = s * _NC + c
    z = jnp.zeros((_L,), jnp.float32)

    def zr(r, carry):
        def zc(cc, carry2):
            rows[0, r, pl.ds(cc * _L, _L)] = z
            return carry2
        return lax.fori_loop(0, _D // _L, zc, carry)

    lax.fori_loop(0, _K, zr, 0)
    base = s * _RPS
    for t in range(5):
        pltpu.sync_copy(rows.at[0], acc_sh.at[pl.ds(base + t * _K, _K)])
    pltpu.sync_copy(rows.at[0, pl.ds(0, 7)],
                    acc_sh.at[pl.ds(base + 5 * _K, 7)])
    pltpu.sync_copy(dst_hbm.at[wid], dstv)
    plsc.subcore_barrier()

    def group(g, carry):
        pltpu.sync_copy(src_hbm.at[wid, g], srcv)
        c0 = g * _G
        pltpu.async_copy(y_hbm.at[srcv.at[0]], rows.at[0], gsem).wait()

        def step(j, carry2):
            jj = jnp.minimum(j, _G - 1)
            d = pltpu.async_copy(y_hbm.at[srcv.at[jj]], rows.at[j & 1], gsem)
            jp = j - 1
            pltpu.sync_copy(rows.at[jp & 1], acc_sh.at[dstv.at[c0 + jp]],
                            add=True)
            d.wait()
            return carry2

        return lax.fori_loop(1, _G + 1, step, carry)

    lax.fori_loop(0, _NG, group, 0)
    plsc.subcore_barrier()
    pltpu.sync_copy(acc_sh.at[pl.ds(base, _RPS)],
                    out_hbm.at[c, pl.ds(base, _RPS)])


_scatter_call = functools.partial(
    pl.kernel,
    out_type=jax.ShapeDtypeStruct((_NC, _NP, _D), jnp.float32),
    mesh=plsc.VectorSubcoreMesh(core_axis_name="c", subcore_axis_name="s"),
    scratch_types=[
        pltpu.VMEM((_G, _K), jnp.int32),
        pltpu.VMEM((_CH, _K), jnp.int32),
        pltpu.VMEM((2, _K, _D), jnp.float32),
        pltpu.SemaphoreType.DMA,
        pltpu.VMEM_SHARED((_NP, _D), jnp.float32),
    ],
)(_scatter_body)


_BLK = 2000


def _k1a_body(x_ref, w_ref, o_ref):
    o_ref[...] = jnp.dot(x_ref[...], w_ref[...],
                         preferred_element_type=jnp.float32)


def _k1b_body(xw_ref, dv_ref, o_ref):
    o_ref[...] = xw_ref[...] * dv_ref[...]


def _k2_body(sa_ref, sb_ref, y_ref, dv_ref, b_ref, w_ref, o_ref):
    pre = (sa_ref[0] + sb_ref[0] + y_ref[...]) * dv_ref[...] + b_ref[...]
    h = jnp.maximum(pre, 0.0)
    hw = jnp.dot(h, w_ref[...], preferred_element_type=jnp.float32)
    o_ref[...] = hw * dv_ref[...]


def _k3_body(sa_ref, sb_ref, y_ref, dv_ref, b_ref, o_ref):
    o_ref[...] = (sa_ref[0] + sb_ref[0] + y_ref[...]) * dv_ref[...] + b_ref[...]


_row_spec = pl.BlockSpec((_BLK, _D), lambda i: (i, 0))
_sa_spec = pl.BlockSpec((1, _BLK, _D), lambda i: (0, i, 0))
_sb_spec = pl.BlockSpec((1, _BLK, _D), lambda i: (1, i, 0))
_col_spec = pl.BlockSpec((_BLK, 1), lambda i: (i, 0))
_w_spec = pl.BlockSpec((_D, _D), lambda i: (0, 0))
_b_spec = pl.BlockSpec((1, _D), lambda i: (0, 0))
_out_sds = jax.ShapeDtypeStruct((_N, _D), jnp.float32)
_GRID = (_N // _BLK,)

_k1a = pl.pallas_call(_k1a_body, grid=_GRID,
                      in_specs=[_row_spec, _w_spec],
                      out_specs=_row_spec, out_shape=_out_sds)
_k1b = pl.pallas_call(_k1b_body, grid=_GRID,
                      in_specs=[_row_spec, _col_spec],
                      out_specs=_row_spec, out_shape=_out_sds)
_k2 = pl.pallas_call(_k2_body, grid=_GRID,
                     in_specs=[_sa_spec, _sb_spec, _row_spec, _col_spec,
                               _b_spec, _w_spec],
                     out_specs=_row_spec, out_shape=_out_sds)
_k3 = pl.pallas_call(_k3_body, grid=_GRID,
                     in_specs=[_sa_spec, _sb_spec, _row_spec, _col_spec,
                               _b_spec],
                     out_specs=_row_spec, out_shape=_out_sds)


def kernel(x, edge_index, W1, b1, W2, b2):
    src = edge_index[0]
    dst = edge_index[1]
    dst_a = dst.reshape(_NS, _EPT1)
    pad = _CH * _K - _EPT

    def _pack(a, padval):
        return jnp.concatenate(
            [a.reshape(_NW, _EPT),
             jnp.full((_NW, pad), padval, jnp.int32)],
            axis=1).reshape(_NW, _CH, _K)

    src_b = _pack(src, 0).reshape(_NW, _NG, _G, _K)
    dst_b = _pack(dst, _TRASH)
    b1r = b1.reshape(1, _D)
    b2r = b2.reshape(1, _D)

    xw1 = _k1a(x, W1)
    dinv = _dinv_call(dst_a)
    dinv_col = dinv[:_N].reshape(_N, 1)
    y1 = _k1b(xw1, dinv_col)
    s1 = _scatter_call(y1, src_b, dst_b)
    y2 = _k2(s1, s1, y1, dinv_col, b1r, W2)
    s2 = _scatter_call(y2, src_b, dst_b)
    out = _k3(s2, s2, y2, dinv_col, b2r)
    return out


# speedup vs baseline: 1.0370x; 1.0370x over previous
import functools

import jax
import jax.numpy as jnp
from jax import lax
from jax.experimental import pallas as pl
from jax.experimental.pallas import tpu as pltpu
from jax.experimental.pallas import tpu_sc as plsc

_NC, _NS, _L = 2, 16, 16
_NW = _NC * _NS

_N = 10000
_E = 320000
_D = 128

_EPT = _E // _NW
_K = 125
_CH = 80
_G = 40
_NG = _CH // _G
_NP = 10112
_RPS = _NP // _NS
_TRASH = _N + 8

_NPAD = 16384
_EPT1 = _E // _NS
_DPS = _NPAD // _NS


def _zero_rows(ref, nrows):
    z = jnp.zeros((_L,), jnp.float32)

    def body(r, carry):
        def inner(c, carry2):
            ref[r, pl.ds(c * _L, _L)] = z
            return carry2
        return lax.fori_loop(0, _D // _L, inner, carry)

    lax.fori_loop(0, nrows, body, 0)


def _rsqrt16(d):
    i = plsc.bitcast(d, jnp.int32)
    i = 0x5F3759DF - (i >> 1)
    y = plsc.bitcast(i, jnp.float32)
    h = 0.5 * d
    y = y * (1.5 - h * y * y)
    y = y * (1.5 - h * y * y)
    y = y * (1.5 - h * y * y)
    return y


def _dinv_body(dst_hbm, dinv_hbm, dstv, degv, redv, acc_sh):
    s = lax.axis_index("s")
    z = jnp.zeros((_L,), jnp.float32)

    def zz(i, carry):
        degv[pl.ds(i * _L, _L)] = z
        return carry

    lax.fori_loop(0, _NPAD // _L, zz, 0)
    pltpu.sync_copy(dst_hbm.at[s], dstv)

    ones = jnp.full((_L,), 1.0, jnp.float32)

    def hist(i, carry):
        idx = dstv[pl.ds(i * _L, _L)]
        plsc.addupdate_scatter(degv, [idx], ones)
        return carry

    lax.fori_loop(0, _EPT1 // _L, hist, 0)
    pltpu.sync_copy(degv, acc_sh.at[s])
    plsc.subcore_barrier()
    base = s * _DPS
    for t in range(_NS):
        pltpu.sync_copy(acc_sh.at[t, pl.ds(base, _DPS)], redv.at[t])

    def red(i, carry):
        acc = redv[0, pl.ds(i * _L, _L)]
        for t in range(1, _NS):
            acc = acc + redv[t, pl.ds(i * _L, _L)]
        degv[pl.ds(i * _L, _L)] = _rsqrt16(acc + 1.0)
        return carry

    lax.fori_loop(0, _DPS // _L, red, 0)
    pltpu.sync_copy(degv.at[pl.ds(0, _DPS)], dinv_hbm.at[pl.ds(base, _DPS)])


_dinv_call = functools.partial(
    pl.kernel,
    out_type=jax.ShapeDtypeStruct((_NPAD,), jnp.float32),
    mesh=plsc.VectorSubcoreMesh(core_axis_name="c", subcore_axis_name="s",
                                num_cores=1),
    scratch_types=[
        pltpu.VMEM((_EPT1,), jnp.int32),
        pltpu.VMEM((_NPAD,), jnp.float32),
        pltpu.VMEM((_NS, _DPS), jnp.float32),
        pltpu.VMEM_SHARED((_NS, _NPAD), jnp.float32),
    ],
    compiler_params=pltpu.CompilerParams(needs_layout_passes=False),
)(_dinv_body)


def _scatter_body(y_hbm, src_hbm, dst_hbm, out_hbm, srcv, dstv, rows, gsem,
                  acc_sh):
    c = lax.axis_index("c")
    s = lax.axis_index("s")
    wid = s * _NC + c
    z = jnp.zeros((_L,), jnp.float32)

    def zr(r, carry):
        def zc(cc, carry2):
            rows[0, r, pl.ds(cc * _L, _L)] = z
            return carry2
        return lax.fori_loop(0, _D // _L, zc, carry)

    lax.fori_loop(0, _K, zr, 0)
    base = s * _RPS
    for t in range(5):
        pltpu.sync_copy(rows.at[0], acc_sh.at[pl.ds(base + t * _K, _K)])
    pltpu.sync_copy(rows.at[0, pl.ds(0, 7)],
                    acc_sh.at[pl.ds(base + 5 * _K, 7)])
    pltpu.sync_copy(dst_hbm.at[wid], dstv)
    plsc.subcore_barrier()

    def group(g, carry):
        pltpu.sync_copy(src_hbm.at[wid, g], srcv)
        c0 = g * _G
        pltpu.async_copy(y_hbm.at[srcv.at[0]], rows.at[0], gsem).wait()

        def step(j, carry2):
            jj = jnp.minimum(j, _G - 1)
            d = pltpu.async_copy(y_hbm.at[srcv.at[jj]], rows.at[j & 1], gsem)
            jp = j - 1
            pltpu.sync_copy(rows.at[jp & 1], acc_sh.at[dstv.at[c0 + jp]],
                            add=True)
            d.wait()
            return carry2

        return lax.fori_loop(1, _G + 1, step, carry)

    lax.fori_loop(0, _NG, group, 0)
    plsc.subcore_barrier()
    pltpu.sync_copy(acc_sh.at[pl.ds(base, _RPS)],
                    out_hbm.at[c, pl.ds(base, _RPS)])


_scatter_call = functools.partial(
    pl.kernel,
    out_type=jax.ShapeDtypeStruct((_NC, _NP, _D), jnp.float32),
    mesh=plsc.VectorSubcoreMesh(core_axis_name="c", subcore_axis_name="s"),
    scratch_types=[
        pltpu.VMEM((_G, _K), jnp.int32),
        pltpu.VMEM((_CH, _K), jnp.int32),
        pltpu.VMEM((2, _K, _D), jnp.float32),
        pltpu.SemaphoreType.DMA,
        pltpu.VMEM_SHARED((_NP, _D), jnp.float32),
    ],
)(_scatter_body)


_BLK = 2000


def _k1_body(x_ref, w_ref, dv_ref, o_ref):
    xw = jnp.dot(x_ref[...], w_ref[...], preferred_element_type=jnp.float32)
    o_ref[...] = xw * dv_ref[...]


def _k2_body(sa_ref, sb_ref, y_ref, dv_ref, b_ref, w_ref, o_ref):
    pre = (sa_ref[0] + sb_ref[0] + y_ref[...]) * dv_ref[...] + b_ref[...]
    h = jnp.maximum(pre, 0.0)
    hw = jnp.dot(h, w_ref[...], preferred_element_type=jnp.float32)
    o_ref[...] = hw * dv_ref[...]


def _k3_body(sa_ref, sb_ref, y_ref, dv_ref, b_ref, o_ref):
    o_ref[...] = (sa_ref[0] + sb_ref[0] + y_ref[...]) * dv_ref[...] + b_ref[...]


_row_spec = pl.BlockSpec((_BLK, _D), lambda i: (i, 0))
_sa_spec = pl.BlockSpec((1, _BLK, _D), lambda i: (0, i, 0))
_sb_spec = pl.BlockSpec((1, _BLK, _D), lambda i: (1, i, 0))
_col_spec = pl.BlockSpec((_BLK, 1), lambda i: (i, 0))
_w_spec = pl.BlockSpec((_D, _D), lambda i: (0, 0))
_b_spec = pl.BlockSpec((1, _D), lambda i: (0, 0))
_out_sds = jax.ShapeDtypeStruct((_N, _D), jnp.float32)
_GRID = (_N // _BLK,)

_k1 = pl.pallas_call(_k1_body, grid=_GRID,
                     in_specs=[_row_spec, _w_spec, _col_spec],
                     out_specs=_row_spec, out_shape=_out_sds)
_k2 = pl.pallas_call(_k2_body, grid=_GRID,
                     in_specs=[_sa_spec, _sb_spec, _row_spec, _col_spec,
                               _b_spec, _w_spec],
                     out_specs=_row_spec, out_shape=_out_sds)
_k3 = pl.pallas_call(_k3_body, grid=_GRID,
                     in_specs=[_sa_spec, _sb_spec, _row_spec, _col_spec,
                               _b_spec],
                     out_specs=_row_spec, out_shape=_out_sds)


def kernel(x, edge_index, W1, b1, W2, b2):
    src = edge_index[0]
    dst = edge_index[1]
    dst_a = dst.reshape(_NS, _EPT1)
    pad = _CH * _K - _EPT

    def _pack(a, padval):
        return jnp.concatenate(
            [a.reshape(_NW, _EPT),
             jnp.full((_NW, pad), padval, jnp.int32)],
            axis=1).reshape(_NW, _CH, _K)

    src_b = _pack(src, 0).reshape(_NW, _NG, _G, _K)
    dst_b = _pack(dst, _TRASH)
    b1r = b1.reshape(1, _D)
    b2r = b2.reshape(1, _D)

    dinv = _dinv_call(dst_a)
    dinv_col = dinv[:_N].reshape(_N, 1)
    y1 = _k1(x, W1, dinv_col)
    s1 = _scatter_call(y1, src_b, dst_b)
    y2 = _k2(s1, s1, y1, dinv_col, b1r, W2)
    s2 = _scatter_call(y2, src_b, dst_b)
    out = _k3(s2, s2, y2, dinv_col, b2r)
    return out


# speedup vs baseline: 1.0384x; 1.0014x over previous
import functools

import jax
import jax.numpy as jnp
from jax import lax
from jax.experimental import pallas as pl
from jax.experimental.pallas import tpu as pltpu
from jax.experimental.pallas import tpu_sc as plsc

_NC, _NS, _L = 2, 16, 16
_NW = _NC * _NS

_N = 10000
_E = 320000
_D = 128

_EPT = _E // _NW
_K = 125
_CH = 80
_G = 40
_NG = _CH // _G
_NP = 10112
_RPS = _NP // _NS
_TRASH = _N + 8

_NPAD = 16384
_EPT1 = _E // _NS
_DPS = _NPAD // _NS


def _zero_rows(ref, nrows):
    z = jnp.zeros((_L,), jnp.float32)

    def body(r, carry):
        def inner(c, carry2):
            ref[r, pl.ds(c * _L, _L)] = z
            return carry2
        return lax.fori_loop(0, _D // _L, inner, carry)

    lax.fori_loop(0, nrows, body, 0)


def _rsqrt16(d):
    i = plsc.bitcast(d, jnp.int32)
    i = 0x5F3759DF - (i >> 1)
    y = plsc.bitcast(i, jnp.float32)
    h = 0.5 * d
    y = y * (1.5 - h * y * y)
    y = y * (1.5 - h * y * y)
    y = y * (1.5 - h * y * y)
    return y


def _dinv_body(dst_hbm, dinv_hbm, dstv, degv, redv, acc_sh):
    s = lax.axis_index("s")
    z = jnp.zeros((_L,), jnp.float32)

    def zz(i, carry):
        degv[pl.ds(i * _L, _L)] = z
        return carry

    lax.fori_loop(0, _NPAD // _L, zz, 0)
    pltpu.sync_copy(dst_hbm.at[s], dstv)

    ones = jnp.full((_L,), 1.0, jnp.float32)

    def hist(i, carry):
        idx = dstv[pl.ds(i * _L, _L)]
        plsc.addupdate_scatter(degv, [idx], ones)
        return carry

    lax.fori_loop(0, _EPT1 // _L, hist, 0)
    pltpu.sync_copy(degv, acc_sh.at[s])
    plsc.subcore_barrier()
    base = s * _DPS
    for t in range(_NS):
        pltpu.sync_copy(acc_sh.at[t, pl.ds(base, _DPS)], redv.at[t])

    def red(i, carry):
        acc = redv[0, pl.ds(i * _L, _L)]
        for t in range(1, _NS):
            acc = acc + redv[t, pl.ds(i * _L, _L)]
        degv[pl.ds(i * _L, _L)] = _rsqrt16(acc + 1.0)
        return carry

    lax.fori_loop(0, _DPS // _L, red, 0)
    pltpu.sync_copy(degv.at[pl.ds(0, _DPS)], dinv_hbm.at[pl.ds(base, _DPS)])


_dinv_call = functools.partial(
    pl.kernel,
    out_type=jax.ShapeDtypeStruct((_NPAD,), jnp.float32),
    mesh=plsc.VectorSubcoreMesh(core_axis_name="c", subcore_axis_name="s",
                                num_cores=1),
    scratch_types=[
        pltpu.VMEM((_EPT1,), jnp.int32),
        pltpu.VMEM((_NPAD,), jnp.float32),
        pltpu.VMEM((_NS, _DPS), jnp.float32),
        pltpu.VMEM_SHARED((_NS, _NPAD), jnp.float32),
    ],
    compiler_params=pltpu.CompilerParams(needs_layout_passes=False),
)(_dinv_body)


def _scatter_body(y_hbm, src_hbm, dst_hbm, out_hbm, srcv, dstv, rows, gsem,
                  acc_sh):
    c = lax.axis_index("c")
    s = lax.axis_index("s")
    wid = s * _NC + c
    z = jnp.zeros((_L,), jnp.float32)

    def zr(r, carry):
        def zc(cc, carry2):
            rows[0, r, pl.ds(cc * _L, _L)] = z
            return carry2
        return lax.fori_loop(0, _D // _L, zc, carry)

    lax.fori_loop(0, _K, zr, 0)
    base = s * _RPS
    for t in range(5):
        pltpu.sync_copy(rows.at[0], acc_sh.at[pl.ds(base + t * _K, _K)])
    pltpu.sync_copy(rows.at[0, pl.ds(0, 7)],
                    acc_sh.at[pl.ds(base + 5 * _K, 7)])
    pltpu.sync_copy(dst_hbm.at[wid], dstv)
    plsc.subcore_barrier()

    def group(g, carry):
        pltpu.sync_copy(src_hbm.at[wid, g], srcv)
        c0 = g * _G

        def pair(p, carry2):
            a = 2 * p
            b = 2 * p + 1
            da = pltpu.async_copy(y_hbm.at[srcv.at[a]], rows.at[0], gsem)
            db = pltpu.async_copy(y_hbm.at[srcv.at[b]], rows.at[1], gsem)
            da.wait()
            pltpu.sync_copy(rows.at[0], acc_sh.at[dstv.at[c0 + a]], add=True)
            db.wait()
            pltpu.sync_copy(rows.at[1], acc_sh.at[dstv.at[c0 + b]], add=True)
            return carry2

        return lax.fori_loop(0, _G // 2, pair, carry)

    lax.fori_loop(0, _NG, group, 0)
    plsc.subcore_barrier()
    pltpu.sync_copy(acc_sh.at[pl.ds(base, _RPS)],
                    out_hbm.at[c, pl.ds(base, _RPS)])


_scatter_call = functools.partial(
    pl.kernel,
    out_type=jax.ShapeDtypeStruct((_NC, _NP, _D), jnp.float32),
    mesh=plsc.VectorSubcoreMesh(core_axis_name="c", subcore_axis_name="s"),
    scratch_types=[
        pltpu.VMEM((_G, _K), jnp.int32),
        pltpu.VMEM((_CH, _K), jnp.int32),
        pltpu.VMEM((2, _K, _D), jnp.float32),
        pltpu.SemaphoreType.DMA,
        pltpu.VMEM_SHARED((_NP, _D), jnp.float32),
    ],
)(_scatter_body)


_BLK = 2000


def _k1_body(x_ref, w_ref, dv_ref, o_ref):
    xw = jnp.dot(x_ref[...], w_ref[...], preferred_element_type=jnp.float32)
    o_ref[...] = xw * dv_ref[...]


def _k2_body(sa_ref, sb_ref, y_ref, dv_ref, b_ref, w_ref, o_ref):
    pre = (sa_ref[0] + sb_ref[0] + y_ref[...]) * dv_ref[...] + b_ref[...]
    h = jnp.maximum(pre, 0.0)
    hw = jnp.dot(h, w_ref[...], preferred_element_type=jnp.float32)
    o_ref[...] = hw * dv_ref[...]


def _k3_body(sa_ref, sb_ref, y_ref, dv_ref, b_ref, o_ref):
    o_ref[...] = (sa_ref[0] + sb_ref[0] + y_ref[...]) * dv_ref[...] + b_ref[...]


_row_spec = pl.BlockSpec((_BLK, _D), lambda i: (i, 0))
_sa_spec = pl.BlockSpec((1, _BLK, _D), lambda i: (0, i, 0))
_sb_spec = pl.BlockSpec((1, _BLK, _D), lambda i: (1, i, 0))
_col_spec = pl.BlockSpec((_BLK, 1), lambda i: (i, 0))
_w_spec = pl.BlockSpec((_D, _D), lambda i: (0, 0))
_b_spec = pl.BlockSpec((1, _D), lambda i: (0, 0))
_out_sds = jax.ShapeDtypeStruct((_N, _D), jnp.float32)
_GRID = (_N // _BLK,)

_k1 = pl.pallas_call(_k1_body, grid=_GRID,
                     in_specs=[_row_spec, _w_spec, _col_spec],
                     out_specs=_row_spec, out_shape=_out_sds)
_k2 = pl.pallas_call(_k2_body, grid=_GRID,
                     in_specs=[_sa_spec, _sb_spec, _row_spec, _col_spec,
                               _b_spec, _w_spec],
                     out_specs=_row_spec, out_shape=_out_sds)
_k3 = pl.pallas_call(_k3_body, grid=_GRID,
                     in_specs=[_sa_spec, _sb_spec, _row_spec, _col_spec,
                               _b_spec],
                     out_specs=_row_spec, out_shape=_out_sds)


def kernel(x, edge_index, W1, b1, W2, b2):
    src = edge_index[0]
    dst = edge_index[1]
    dst_a = dst.reshape(_NS, _EPT1)
    pad = _CH * _K - _EPT

    def _pack(a, padval):
        return jnp.concatenate(
            [a.reshape(_NW, _EPT),
             jnp.full((_NW, pad), padval, jnp.int32)],
            axis=1).reshape(_NW, _CH, _K)

    src_b = _pack(src, 0).reshape(_NW, _NG, _G, _K)
    dst_b = _pack(dst, _TRASH)
    b1r = b1.reshape(1, _D)
    b2r = b2.reshape(1, _D)

    dinv = _dinv_call(dst_a)
    dinv_col = dinv[:_N].reshape(_N, 1)
    y1 = _k1(x, W1, dinv_col)
    s1 = _scatter_call(y1, src_b, dst_b)
    y2 = _k2(s1, s1, y1, dinv_col, b1r, W2)
    s2 = _scatter_call(y2, src_b, dst_b)
    out = _k3(s2, s2, y2, dinv_col, b2r)
    return out
